# baseline (speedup 1.0000x reference)
import functools

import jax
import jax.numpy as jnp
from jax import lax
from jax.experimental import pallas as pl
from jax.experimental.pallas import tpu as pltpu
from jax.experimental.pallas import tpu_sc as plsc

N = 320000
D = 128
G = 64
NUM_SEGMENTS = 256

GRP = 128
NGRP = N // GRP
CH = 4
CGRP = NGRP // CH
CROWS = N // CH
ROW_BLK = 2000
CBLK = CROWS // ROW_BLK
NW = 32
MAXG = CGRP // NW + 1


def _pre_ffn_body(x_ref, w1_ref, b1_ref, w2_ref, b2_ref, out_ref):
    xb = x_ref[...].astype(jnp.bfloat16)
    h = jnp.maximum(
        lax.dot_general(xb, w1_ref[...], (((1,), (0,)), ((), ())),
                        preferred_element_type=jnp.float32) + b1_ref[...],
        0.0)
    out_ref[...] = lax.dot_general(h.astype(jnp.bfloat16), w2_ref[...],
                                   (((1,), (0,)), ((), ())),
                                   preferred_element_type=jnp.float32) + b2_ref[...]


def _post_ffn_body(p_ref, w1_ref, b1_ref, w2_ref, b2_ref, out_ref):
    g = jnp.sum(p_ref[...], axis=0)
    h = jnp.maximum(
        lax.dot_general(g, w1_ref[...], (((1,), (0,)), ((), ())),
                        preferred_element_type=jnp.float32) + b1_ref[...],
        0.0)
    out_ref[...] = lax.dot_general(h, w2_ref[...], (((1,), (0,)), ((), ())),
                                   preferred_element_type=jnp.float32) + b2_ref[...]


def _sc_chunk_body(idx3, ftx, out0, out1, idx_v, rows_v, zeros_v, acc_sh, semi,
                   semg0, semg1, semg2, semg3, sems0, sems1, sems2, sems3):
    nc = lax.axis_size("c")
    ns = lax.axis_size("s")
    c = lax.axis_index("c")
    s = lax.axis_index("s")
    w = c * ns + s
    semg = (semg0, semg1, semg2, semg3)
    semsc = (sems0, sems1, sems2, sems3)

    g0 = (CGRP * w) // NW
    g1 = (CGRP * (w + 1)) // NW
    ng = g1 - g0

    pltpu.async_copy(idx3.at[w], idx_v, semi)

    for i in range(16):
        for j in range(D // 16):
            zeros_v[i, pl.ds(j * 16, 16)] = jnp.zeros((16,), jnp.float32)
    pltpu.sync_copy(zeros_v, acc_sh.at[pl.ds(s * 16, 16)])
    pltpu.make_async_copy(idx3.at[0], idx_v, semi).wait()
    plsc.subcore_barrier()

    def start_rows(j, b):
        pltpu.async_copy(ftx.at[pl.ds((g0 + j) * GRP, GRP)], rows_v.at[b], semg[b])

    def wait_rows(b):
        pltpu.make_async_copy(ftx.at[pl.ds(0, GRP)], rows_v.at[b], semg[b]).wait()

    def start_scat(j, b):
        pltpu.async_copy(rows_v.at[b], acc_sh.at[idx_v.at[j]], semsc[b], add=True)

    def wait_scat(b):
        pltpu.make_async_copy(rows_v.at[b], acc_sh.at[pl.ds(0, GRP)], semsc[b]).wait()

    for j0 in range(2):
        @pl.when(j0 < ng)
        def _():
            start_rows(j0, j0)

    def quad_body(q, carry):
        for b in range(4):
            j = 4 * q + b

            @pl.when(j < ng)
            def _():
                wait_rows(b)
                start_scat(j, b)
                b2 = (b + 2) % 4

                @pl.when(j + 2 < ng)
                def _():
                    @pl.when(j >= 2)
                    def _():
                        wait_scat(b2)
                    start_rows(j + 2, b2)
        return carry

    lax.fori_loop(0, (ng + 3) // 4, quad_body, 0)

    for b in range(4):
        @pl.when((ng >= 4) | (b < ng))
        def _():
            wait_scat(b)
    plsc.subcore_barrier()

    @pl.when((s == 0) & (c == 0))
    def _():
        pltpu.sync_copy(acc_sh, out0)

    @pl.when((s == 0) & (c == 1))
    def _():
        pltpu.sync_copy(acc_sh, out1)


def kernel(x, batch, W1_pre, b1_pre, W2_pre, b2_pre, W1_post, b1_post, W2_post, b2_post):
    batch2d = batch.astype(jnp.int32).reshape(NGRP, GRP)
    b1p = b1_pre.reshape(1, D)
    b2p = b2_pre.reshape(1, D)
    b1q = b1_post.reshape(1, D)
    b2q = b2_post.reshape(1, G)
    w1_bf = W1_pre.astype(jnp.bfloat16)
    w2_bf = W2_pre.astype(jnp.bfloat16)

    g0s = (CGRP * jnp.arange(NW, dtype=jnp.int32)) // NW
    rows = jnp.minimum(g0s[:, None] + jnp.arange(MAXG, dtype=jnp.int32)[None, :],
                       CGRP - 1)
    rows = rows[None, :, :] + CGRP * jnp.arange(CH, dtype=jnp.int32)[:, None, None]
    idx3 = batch2d[rows.reshape(-1)].reshape(CH, NW, MAXG, GRP)

    mesh = plsc.VectorSubcoreMesh(core_axis_name="c", subcore_axis_name="s",
                                  num_cores=2, num_subcores=16)
    sc_call = pl.kernel(
        _sc_chunk_body,
        out_type=[jax.ShapeDtypeStruct((NUM_SEGMENTS, D), jnp.float32),
                  jax.ShapeDtypeStruct((NUM_SEGMENTS, D), jnp.float32)],
        mesh=mesh,
        scratch_types=[
            pltpu.VMEM((MAXG, GRP), jnp.int32),
            pltpu.VMEM((4, GRP, D), jnp.float32),
            pltpu.VMEM((16, D), jnp.float32),
            pltpu.VMEM_SHARED((NUM_SEGMENTS, D), jnp.float32),
        ] + [pltpu.SemaphoreType.DMA] * 9,
    )

    partials = []
    for k in range(CH):
        ftx_k = pl.pallas_call(

            _pre_ffn_body,
            grid=(CBLK,),
            in_specs=[
                pl.BlockSpec((ROW_BLK, D), functools.partial(lambda k_, i: (k_ * CBLK + i, 0), k)),
                pl.BlockSpec((D, D), lambda i: (0, 0)),
                pl.BlockSpec((1, D), lambda i: (0, 0)),
                pl.BlockSpec((D, D), lambda i: (0, 0)),
                pl.BlockSpec((1, D), lambda i: (0, 0)),
            ],
            out_specs=pl.BlockSpec((ROW_BLK, D), lambda i: (i, 0)),
            out_shape=jax.ShapeDtypeStruct((CROWS, D), jnp.float32),
        )(x, w1_bf, b1p, w2_bf, b2p)
        p0, p1 = sc_call(idx3[k], ftx_k)
        partials.append(p0)
        partials.append(p1)

    pstack = jnp.stack(partials, axis=0)

    out = pl.pallas_call(
        _post_ffn_body,
        in_specs=[
            pl.BlockSpec((2 * CH, NUM_SEGMENTS, D), lambda: (0, 0, 0)),
            pl.BlockSpec((D, D), lambda: (0, 0)),
            pl.BlockSpec((1, D), lambda: (0, 0)),
            pl.BlockSpec((D, G), lambda: (0, 0)),
            pl.BlockSpec((1, G), lambda: (0, 0)),
        ],
        out_specs=pl.BlockSpec((NUM_SEGMENTS, G), lambda: (0, 0)),
        out_shape=jax.ShapeDtypeStruct((NUM_SEGMENTS, G), jnp.float32),
    )(pstack, W1_post, b1q, W2_post, b2q)
    return out


# speedup vs baseline: 1.1878x; 1.1878x over previous
import functools

import jax
import jax.numpy as jnp
from jax import lax
from jax.experimental import pallas as pl
from jax.experimental.pallas import tpu as pltpu
from jax.experimental.pallas import tpu_sc as plsc

N = 320000
D = 128
G = 64
NUM_SEGMENTS = 256

GRP = 128
NGRP = N // GRP
CH = 4
CGRP = NGRP // CH
CROWS = N // CH
ROW_BLK = 3200
GPB = ROW_BLK // GRP
CBLK = CROWS // ROW_BLK
NW = 32
MAXG = CGRP // NW + 1
PADG = 32
TRASH = NUM_SEGMENTS


def _pre_ffn_body(x_ref, w1_ref, b1_ref, w2_ref, b2_ref, out_ref, gs_ref):
    xb = x_ref[...].astype(jnp.bfloat16)
    h = jnp.maximum(
        lax.dot_general(xb, w1_ref[...], (((1,), (0,)), ((), ())),
                        preferred_element_type=jnp.float32) + b1_ref[...],
        0.0)
    ftx = lax.dot_general(h.astype(jnp.bfloat16), w2_ref[...],
                          (((1,), (0,)), ((), ())),
                          preferred_element_type=jnp.float32) + b2_ref[...]
    out_ref[...] = ftx
    gs_ref[0] = jnp.sum(ftx.reshape(GPB, GRP, D), axis=1)


def _post_ffn_body(p_ref, w1_ref, b1_ref, w2_ref, b2_ref, out_ref):
    g = jnp.sum(p_ref[...], axis=0)
    h = jnp.maximum(
        lax.dot_general(g, w1_ref[...], (((1,), (0,)), ((), ())),
                        preferred_element_type=jnp.float32) + b1_ref[...],
        0.0)
    out_ref[...] = lax.dot_general(h, w2_ref[...], (((1,), (0,)), ((), ())),
                                   preferred_element_type=jnp.float32) + b2_ref[...]


def _sc_chunk_body(idx3, ftx, gsums, tgt3, bnd3, out0, out1,
                   idx_v, gsum_v, tgt_v, bnd_v, rows_b, zeros_v, acc_sh,
                   semi, semg, semt):
    nc = lax.axis_size("c")
    ns = lax.axis_size("s")
    c = lax.axis_index("c")
    s = lax.axis_index("s")
    w = c * ns + s

    g0 = (CGRP * w) // NW
    g1 = (CGRP * (w + 1)) // NW
    ng = g1 - g0

    pltpu.async_copy(idx3.at[w], idx_v.at[pl.ds(0, MAXG)], semi)
    pltpu.async_copy(gsums.at[w], gsum_v, semg)
    pltpu.async_copy(tgt3.at[w], tgt_v, semt)
    pltpu.async_copy(bnd3.at[w], bnd_v, semt)

    for i in range(16):
        for j in range(D // 16):
            zeros_v[i, pl.ds(j * 16, 16)] = jnp.zeros((16,), jnp.float32)
    pltpu.sync_copy(zeros_v, acc_sh.at[pl.ds(s * 16, 16)])

    @pl.when(s == 0)
    def _():
        pltpu.sync_copy(zeros_v.at[pl.ds(0, 1)], acc_sh.at[pl.ds(256, 1)])

    pltpu.make_async_copy(idx3.at[0], idx_v.at[pl.ds(0, MAXG)], semi).wait()
    pltpu.make_async_copy(tgt3.at[0], tgt_v, semt).wait()
    pltpu.make_async_copy(bnd3.at[0], bnd_v, semt).wait()
    pltpu.make_async_copy(gsums.at[0], gsum_v, semg).wait()
    plsc.subcore_barrier()

    pltpu.sync_copy(gsum_v, acc_sh.at[tgt_v], add=True)

    def body(j, carry):
        flag = bnd_v[pl.ds(j, 16)][0]

        @pl.when(flag > 0)
        def _():
            pltpu.sync_copy(ftx.at[pl.ds((g0 + j) * GRP, GRP)], rows_b)
            pltpu.sync_copy(rows_b, acc_sh.at[idx_v.at[j]], add=True)
        return carry

    lax.fori_loop(0, ng, body, 0)
    plsc.subcore_barrier()

    @pl.when((s == 0) & (c == 0))
    def _():
        pltpu.sync_copy(acc_sh.at[pl.ds(0, NUM_SEGMENTS)], out0)

    @pl.when((s == 0) & (c == 1))
    def _():
        pltpu.sync_copy(acc_sh.at[pl.ds(0, NUM_SEGMENTS)], out1)


def kernel(x, batch, W1_pre, b1_pre, W2_pre, b2_pre, W1_post, b1_post, W2_post, b2_post):
    batch2d = batch.astype(jnp.int32).reshape(NGRP, GRP)
    b1p = b1_pre.reshape(1, D)
    b2p = b2_pre.reshape(1, D)
    b1q = b1_post.reshape(1, D)
    b2q = b2_post.reshape(1, G)
    w1_bf = W1_pre.astype(jnp.bfloat16)
    w2_bf = W2_pre.astype(jnp.bfloat16)

    g0s = (CGRP * jnp.arange(NW, dtype=jnp.int32)) // NW
    rows = jnp.minimum(g0s[:, None] + jnp.arange(MAXG, dtype=jnp.int32)[None, :],
                       CGRP - 1)
    rows = rows[None, :, :] + CGRP * jnp.arange(CH, dtype=jnp.int32)[:, None, None]
    idx3 = batch2d[rows.reshape(-1)].reshape(CH, NW, MAXG, GRP)
    slabrows = jnp.minimum(
        g0s[:, None] + jnp.arange(PADG, dtype=jnp.int32)[None, :], CGRP - 1)
    firsts = batch2d[:, 0]
    lasts = batch2d[:, GRP - 1]
    uni = firsts == lasts
    tgt_global = jnp.where(uni, firsts, TRASH).astype(jnp.int32)
    g1s = (CGRP * (jnp.arange(NW, dtype=jnp.int32) + 1)) // NW
    validP = (jnp.arange(PADG, dtype=jnp.int32)[None, :]
              < (g1s - g0s)[:, None])
    crows = (slabrows[None, :, :]
             + CGRP * jnp.arange(CH, dtype=jnp.int32)[:, None, None])
    tgt3 = jnp.where(validP[None], tgt_global[crows.reshape(-1)].reshape(
        CH, NW, PADG), TRASH).astype(jnp.int32)
    bnd3 = jnp.where(validP[None] & ~uni[crows.reshape(-1)].reshape(
        CH, NW, PADG), 1, 0).astype(jnp.int32)
    bnd3 = jnp.concatenate(
        [bnd3, jnp.zeros((CH, NW, 16), jnp.int32)], axis=-1)

    mesh = plsc.VectorSubcoreMesh(core_axis_name="c", subcore_axis_name="s",
                                  num_cores=2, num_subcores=16)
    sc_call = pl.kernel(
        _sc_chunk_body,
        out_type=[jax.ShapeDtypeStruct((NUM_SEGMENTS, D), jnp.float32),
                  jax.ShapeDtypeStruct((NUM_SEGMENTS, D), jnp.float32)],
        mesh=mesh,
        scratch_types=[
            pltpu.VMEM((PADG, GRP), jnp.int32),
            pltpu.VMEM((PADG, D), jnp.float32),
            pltpu.VMEM((PADG,), jnp.int32),
            pltpu.VMEM((PADG + 16,), jnp.int32),
            pltpu.VMEM((GRP, D), jnp.float32),
            pltpu.VMEM((16, D), jnp.float32),
            pltpu.VMEM_SHARED((NUM_SEGMENTS + 1, D), jnp.float32),
            pltpu.SemaphoreType.DMA,
            pltpu.SemaphoreType.DMA,
            pltpu.SemaphoreType.DMA,
        ],
    )

    partials = []
    for k in range(CH):
        ftx_k, gs_k = pl.pallas_call(
            _pre_ffn_body,
            grid=(CBLK,),
            in_specs=[
                pl.BlockSpec((ROW_BLK, D), functools.partial(lambda k_, i: (k_ * CBLK + i, 0), k)),
                pl.BlockSpec((D, D), lambda i: (0, 0)),
                pl.BlockSpec((1, D), lambda i: (0, 0)),
                pl.BlockSpec((D, D), lambda i: (0, 0)),
                pl.BlockSpec((1, D), lambda i: (0, 0)),
            ],
            out_specs=[
                pl.BlockSpec((ROW_BLK, D), lambda i: (i, 0)),
                pl.BlockSpec((1, GPB, D), lambda i: (i, 0, 0)),
            ],
            out_shape=[
                jax.ShapeDtypeStruct((CROWS, D), jnp.float32),
                jax.ShapeDtypeStruct((CBLK, GPB, D), jnp.float32),
            ],
        )(x, w1_bf, b1p, w2_bf, b2p)
        gs_flat = gs_k.reshape(CGRP, D)
        gs3 = gs_flat[slabrows.reshape(-1)].reshape(NW, PADG, D)
        p0, p1 = sc_call(idx3[k], ftx_k, gs3, tgt3[k], bnd3[k])
        partials.append(p0)
        partials.append(p1)

    pstack = jnp.stack(partials, axis=0)

    out = pl.pallas_call(
        _post_ffn_body,
        in_specs=[
            pl.BlockSpec((2 * CH, NUM_SEGMENTS, D), lambda: (0, 0, 0)),
            pl.BlockSpec((D, D), lambda: (0, 0)),
            pl.BlockSpec((1, D), lambda: (0, 0)),
            pl.BlockSpec((D, G), lambda: (0, 0)),
            pl.BlockSpec((1, G), lambda: (0, 0)),
        ],
        out_specs=pl.BlockSpec((NUM_SEGMENTS, G), lambda: (0, 0)),
        out_shape=jax.ShapeDtypeStruct((NUM_SEGMENTS, G), jnp.float32),
    )(pstack, W1_post, b1q, W2_post, b2q)
    return out


# speedup vs baseline: 1.2376x; 1.0419x over previous
import functools

import jax
import jax.numpy as jnp
from jax import lax
from jax.experimental import pallas as pl
from jax.experimental.pallas import tpu as pltpu
from jax.experimental.pallas import tpu_sc as plsc

N = 320000
D = 128
G = 64
NUM_SEGMENTS = 256

GRP = 128
NGRP = N // GRP
CH = 4
CGRP = NGRP // CH
CROWS = N // CH
ROW_BLK = 3200
GPB = ROW_BLK // GRP
CBLK = CROWS // ROW_BLK
NW = 32
MAXG = CGRP // NW + 1
PADG = 32
TRASH = NUM_SEGMENTS


def _pre_ffn_body(x_ref, w1_ref, b1_ref, w2_ref, b2_ref, out_ref, gs_ref):
    xb = x_ref[...].astype(jnp.bfloat16)
    h = jnp.maximum(
        lax.dot_general(xb, w1_ref[...], (((1,), (0,)), ((), ())),
                        preferred_element_type=jnp.float32) + b1_ref[...],
        0.0)
    ftx = lax.dot_general(h.astype(jnp.bfloat16), w2_ref[...],
                          (((1,), (0,)), ((), ())),
                          preferred_element_type=jnp.float32) + b2_ref[...]
    out_ref[...] = ftx
    gs_ref[0] = jnp.sum(ftx.reshape(GPB, GRP, D), axis=1)


def _post_ffn_body(p_ref, w1_ref, b1_ref, w2_ref, b2_ref, out_ref):
    g = jnp.sum(p_ref[...], axis=0)
    h = jnp.maximum(
        lax.dot_general(g, w1_ref[...], (((1,), (0,)), ((), ())),
                        preferred_element_type=jnp.float32) + b1_ref[...],
        0.0)
    out_ref[...] = lax.dot_general(h, w2_ref[...], (((1,), (0,)), ((), ())),
                                   preferred_element_type=jnp.float32) + b2_ref[...]


def _sc_chunk_body(idx3, ftx, gsums, tgt3, bnd3, out0, out1,
                   idx_v, gsum_v, tgt_v, bnd_v, rows_b, zeros_v, acc_sh,
                   semi, semg, semt):
    nc = lax.axis_size("c")
    ns = lax.axis_size("s")
    c = lax.axis_index("c")
    s = lax.axis_index("s")
    w = c * ns + s

    g0a = pl.multiple_of(((CGRP * w) // NW // 8) * 8, 8)

    pltpu.async_copy(idx3.at[w], idx_v, semi)
    pltpu.async_copy(gsums.at[pl.ds(g0a, PADG)], gsum_v, semg)
    pltpu.async_copy(tgt3.at[w], tgt_v, semt)
    pltpu.async_copy(bnd3.at[w], bnd_v, semt)

    for i in range(16):
        for j in range(D // 16):
            zeros_v[i, pl.ds(j * 16, 16)] = jnp.zeros((16,), jnp.float32)
    pltpu.sync_copy(zeros_v, acc_sh.at[pl.ds(s * 16, 16)])

    @pl.when(s == 0)
    def _():
        pltpu.sync_copy(zeros_v.at[pl.ds(0, 1)], acc_sh.at[pl.ds(256, 1)])

    pltpu.make_async_copy(idx3.at[0], idx_v, semi).wait()
    pltpu.make_async_copy(tgt3.at[0], tgt_v, semt).wait()
    pltpu.make_async_copy(bnd3.at[0], bnd_v, semt).wait()
    pltpu.make_async_copy(gsums.at[pl.ds(0, PADG)], gsum_v, semg).wait()
    plsc.subcore_barrier()

    pltpu.sync_copy(gsum_v, acc_sh.at[tgt_v], add=True)

    def body(j, carry):
        flag = bnd_v[pl.ds(j, 16)][0]

        @pl.when(flag > 0)
        def _():
            pltpu.sync_copy(ftx.at[pl.ds((g0a + j) * GRP, GRP)], rows_b)
            pltpu.sync_copy(rows_b, acc_sh.at[idx_v.at[j]], add=True)
        return carry

    lax.fori_loop(0, PADG, body, 0)
    plsc.subcore_barrier()

    @pl.when((s == 0) & (c == 0))
    def _():
        pltpu.sync_copy(acc_sh.at[pl.ds(0, NUM_SEGMENTS)], out0)

    @pl.when((s == 0) & (c == 1))
    def _():
        pltpu.sync_copy(acc_sh.at[pl.ds(0, NUM_SEGMENTS)], out1)


def kernel(x, batch, W1_pre, b1_pre, W2_pre, b2_pre, W1_post, b1_post, W2_post, b2_post):
    batch2d = batch.astype(jnp.int32).reshape(NGRP, GRP)
    b1p = b1_pre.reshape(1, D)
    b2p = b2_pre.reshape(1, D)
    b1q = b1_post.reshape(1, D)
    b2q = b2_post.reshape(1, G)
    w1_bf = W1_pre.astype(jnp.bfloat16)
    w2_bf = W2_pre.astype(jnp.bfloat16)

    g0s = (CGRP * jnp.arange(NW, dtype=jnp.int32)) // NW
    g1s = (CGRP * (jnp.arange(NW, dtype=jnp.int32) + 1)) // NW
    g0as = (g0s // 8) * 8
    slabrows = jnp.minimum(
        g0as[:, None] + jnp.arange(PADG, dtype=jnp.int32)[None, :], CGRP - 1)
    crows = (slabrows[None, :, :]
             + CGRP * jnp.arange(CH, dtype=jnp.int32)[:, None, None])
    idx3 = batch2d[crows.reshape(-1)].reshape(CH, NW, PADG, GRP)
    firsts = batch2d[:, 0]
    lasts = batch2d[:, GRP - 1]
    uni = firsts == lasts
    tgt_global = jnp.where(uni, firsts, TRASH).astype(jnp.int32)
    slot = jnp.arange(PADG, dtype=jnp.int32)[None, :]
    validP = ((slot >= (g0s - g0as)[:, None])
              & (slot < (g1s - g0as)[:, None]))
    tgt3 = jnp.where(validP[None], tgt_global[crows.reshape(-1)].reshape(
        CH, NW, PADG), TRASH).astype(jnp.int32)
    bnd3 = jnp.where(validP[None] & ~uni[crows.reshape(-1)].reshape(
        CH, NW, PADG), 1, 0).astype(jnp.int32)
    bnd3 = jnp.concatenate(
        [bnd3, jnp.zeros((CH, NW, 16), jnp.int32)], axis=-1)

    mesh = plsc.VectorSubcoreMesh(core_axis_name="c", subcore_axis_name="s",
                                  num_cores=2, num_subcores=16)
    sc_call = pl.kernel(
        _sc_chunk_body,
        out_type=[jax.ShapeDtypeStruct((NUM_SEGMENTS, D), jnp.float32),
                  jax.ShapeDtypeStruct((NUM_SEGMENTS, D), jnp.float32)],
        mesh=mesh,
        scratch_types=[
            pltpu.VMEM((PADG, GRP), jnp.int32),
            pltpu.VMEM((PADG, D), jnp.float32),
            pltpu.VMEM((PADG,), jnp.int32),
            pltpu.VMEM((PADG + 16,), jnp.int32),
            pltpu.VMEM((GRP, D), jnp.float32),
            pltpu.VMEM((16, D), jnp.float32),
            pltpu.VMEM_SHARED((NUM_SEGMENTS + 1, D), jnp.float32),
            pltpu.SemaphoreType.DMA,
            pltpu.SemaphoreType.DMA,
            pltpu.SemaphoreType.DMA,
        ],
    )

    partials = []
    for k in range(CH):
        ftx_k, gs_k = pl.pallas_call(
            _pre_ffn_body,
            grid=(CBLK,),
            in_specs=[
                pl.BlockSpec((ROW_BLK, D), functools.partial(lambda k_, i: (k_ * CBLK + i, 0), k)),
                pl.BlockSpec((D, D), lambda i: (0, 0)),
                pl.BlockSpec((1, D), lambda i: (0, 0)),
                pl.BlockSpec((D, D), lambda i: (0, 0)),
                pl.BlockSpec((1, D), lambda i: (0, 0)),
            ],
            out_specs=[
                pl.BlockSpec((ROW_BLK, D), lambda i: (i, 0)),
                pl.BlockSpec((1, GPB, D), lambda i: (i, 0, 0)),
            ],
            out_shape=[
                jax.ShapeDtypeStruct((CROWS, D), jnp.float32),
                jax.ShapeDtypeStruct((CBLK + 2, GPB, D), jnp.float32),
            ],
        )(x, w1_bf, b1p, w2_bf, b2p)
        gs_flat = gs_k.reshape((CBLK + 2) * GPB, D)
        p0, p1 = sc_call(idx3[k], ftx_k, gs_flat, tgt3[k], bnd3[k])
        partials.append(p0)
        partials.append(p1)

    pstack = jnp.stack(partials, axis=0)

    out = pl.pallas_call(
        _post_ffn_body,
        in_specs=[
            pl.BlockSpec((2 * CH, NUM_SEGMENTS, D), lambda: (0, 0, 0)),
            pl.BlockSpec((D, D), lambda: (0, 0)),
            pl.BlockSpec((1, D), lambda: (0, 0)),
            pl.BlockSpec((D, G), lambda: (0, 0)),
            pl.BlockSpec((1, G), lambda: (0, 0)),
        ],
        out_specs=pl.BlockSpec((NUM_SEGMENTS, G), lambda: (0, 0)),
        out_shape=jax.ShapeDtypeStruct((NUM_SEGMENTS, G), jnp.float32),
    )(pstack, W1_post, b1q, W2_post, b2q)
    return out


# speedup vs baseline: 1.2680x; 1.0246x over previous
import functools

import jax
import jax.numpy as jnp
from jax import lax
from jax.experimental import pallas as pl
from jax.experimental.pallas import tpu as pltpu
from jax.experimental.pallas import tpu_sc as plsc

N = 320000
D = 128
G = 64
NUM_SEGMENTS = 256

GRP = 128
NGRP = N // GRP
CH = 2
CGRP = NGRP // CH
CROWS = N // CH
ROW_BLK = 3200
GPB = ROW_BLK // GRP
CBLK = CROWS // ROW_BLK
NW = 32
PADG = ((CGRP // NW + 1 + 7 + 15) // 16) * 16
TRASH = NUM_SEGMENTS


def _pre_ffn_body(x_ref, w1_ref, b1_ref, w2_ref, b2_ref, out_ref, gs_ref):
    xb = x_ref[...].astype(jnp.bfloat16)
    h = jnp.maximum(
        lax.dot_general(xb, w1_ref[...], (((1,), (0,)), ((), ())),
                        preferred_element_type=jnp.float32) + b1_ref[...],
        0.0)
    ftx = lax.dot_general(h.astype(jnp.bfloat16), w2_ref[...],
                          (((1,), (0,)), ((), ())),
                          preferred_element_type=jnp.float32) + b2_ref[...]
    out_ref[...] = ftx
    gs_ref[0] = jnp.sum(ftx.reshape(GPB, GRP, D), axis=1)


def _post_ffn_body(p_ref, w1_ref, b1_ref, w2_ref, b2_ref, out_ref):
    g = jnp.sum(p_ref[...], axis=0)
    h = jnp.maximum(
        lax.dot_general(g, w1_ref[...], (((1,), (0,)), ((), ())),
                        preferred_element_type=jnp.float32) + b1_ref[...],
        0.0)
    out_ref[...] = lax.dot_general(h, w2_ref[...], (((1,), (0,)), ((), ())),
                                   preferred_element_type=jnp.float32) + b2_ref[...]


def _sc_chunk_body(idx3, ftx, gsums, tgt3, bnd3, out0, out1,
                   idx_v, gsum_v, tgt_v, bnd_v, rows_b, zeros_v, acc_sh,
                   semi, semg, semt):
    nc = lax.axis_size("c")
    ns = lax.axis_size("s")
    c = lax.axis_index("c")
    s = lax.axis_index("s")
    w = c * ns + s

    g0a = pl.multiple_of(((CGRP * w) // NW // 8) * 8, 8)

    pltpu.async_copy(idx3.at[w], idx_v, semi)
    pltpu.async_copy(gsums.at[pl.ds(g0a, PADG)], gsum_v, semg)
    pltpu.async_copy(tgt3.at[w], tgt_v, semt)
    pltpu.async_copy(bnd3.at[w], bnd_v, semt)

    for i in range(16):
        for j in range(D // 16):
            zeros_v[i, pl.ds(j * 16, 16)] = jnp.zeros((16,), jnp.float32)
    pltpu.sync_copy(zeros_v, acc_sh.at[pl.ds(s * 16, 16)])

    @pl.when(s == 0)
    def _():
        pltpu.sync_copy(zeros_v.at[pl.ds(0, 1)], acc_sh.at[pl.ds(256, 1)])

    pltpu.make_async_copy(idx3.at[0], idx_v, semi).wait()
    pltpu.make_async_copy(tgt3.at[0], tgt_v, semt).wait()
    pltpu.make_async_copy(bnd3.at[0], bnd_v, semt).wait()
    pltpu.make_async_copy(gsums.at[pl.ds(0, PADG)], gsum_v, semg).wait()
    plsc.subcore_barrier()

    pltpu.sync_copy(gsum_v, acc_sh.at[tgt_v], add=True)

    def body(j, carry):
        flag = bnd_v[pl.ds(j, 16)][0]

        @pl.when(flag > 0)
        def _():
            pltpu.sync_copy(ftx.at[pl.ds((g0a + j) * GRP, GRP)], rows_b)
            pltpu.sync_copy(rows_b, acc_sh.at[idx_v.at[j]], add=True)
        return carry

    lax.fori_loop(0, PADG, body, 0)
    plsc.subcore_barrier()

    @pl.when((s == 0) & (c == 0))
    def _():
        pltpu.sync_copy(acc_sh.at[pl.ds(0, NUM_SEGMENTS)], out0)

    @pl.when((s == 0) & (c == 1))
    def _():
        pltpu.sync_copy(acc_sh.at[pl.ds(0, NUM_SEGMENTS)], out1)


def kernel(x, batch, W1_pre, b1_pre, W2_pre, b2_pre, W1_post, b1_post, W2_post, b2_post):
    batch2d = batch.astype(jnp.int32).reshape(NGRP, GRP)
    b1p = b1_pre.reshape(1, D)
    b2p = b2_pre.reshape(1, D)
    b1q = b1_post.reshape(1, D)
    b2q = b2_post.reshape(1, G)
    w1_bf = W1_pre.astype(jnp.bfloat16)
    w2_bf = W2_pre.astype(jnp.bfloat16)

    g0s = (CGRP * jnp.arange(NW, dtype=jnp.int32)) // NW
    g1s = (CGRP * (jnp.arange(NW, dtype=jnp.int32) + 1)) // NW
    g0as = (g0s // 8) * 8
    slabrows = jnp.minimum(
        g0as[:, None] + jnp.arange(PADG, dtype=jnp.int32)[None, :], CGRP - 1)
    crows = (slabrows[None, :, :]
             + CGRP * jnp.arange(CH, dtype=jnp.int32)[:, None, None])
    idx3 = batch2d[crows.reshape(-1)].reshape(CH, NW, PADG, GRP)
    firsts = batch2d[:, 0]
    lasts = batch2d[:, GRP - 1]
    uni = firsts == lasts
    tgt_global = jnp.where(uni, firsts, TRASH).astype(jnp.int32)
    slot = jnp.arange(PADG, dtype=jnp.int32)[None, :]
    validP = ((slot >= (g0s - g0as)[:, None])
              & (slot < (g1s - g0as)[:, None]))
    tgt3 = jnp.where(validP[None], tgt_global[crows.reshape(-1)].reshape(
        CH, NW, PADG), TRASH).astype(jnp.int32)
    bnd3 = jnp.where(validP[None] & ~uni[crows.reshape(-1)].reshape(
        CH, NW, PADG), 1, 0).astype(jnp.int32)
    bnd3 = jnp.concatenate(
        [bnd3, jnp.zeros((CH, NW, 16), jnp.int32)], axis=-1)

    mesh = plsc.VectorSubcoreMesh(core_axis_name="c", subcore_axis_name="s",
                                  num_cores=2, num_subcores=16)
    sc_call = pl.kernel(
        _sc_chunk_body,
        out_type=[jax.ShapeDtypeStruct((NUM_SEGMENTS, D), jnp.float32),
                  jax.ShapeDtypeStruct((NUM_SEGMENTS, D), jnp.float32)],
        mesh=mesh,
        scratch_types=[
            pltpu.VMEM((PADG, GRP), jnp.int32),
            pltpu.VMEM((PADG, D), jnp.float32),
            pltpu.VMEM((PADG,), jnp.int32),
            pltpu.VMEM((PADG + 16,), jnp.int32),
            pltpu.VMEM((GRP, D), jnp.float32),
            pltpu.VMEM((16, D), jnp.float32),
            pltpu.VMEM_SHARED((NUM_SEGMENTS + 1, D), jnp.float32),
            pltpu.SemaphoreType.DMA,
            pltpu.SemaphoreType.DMA,
            pltpu.SemaphoreType.DMA,
        ],
    )

    partials = []
    for k in range(CH):
        ftx_k, gs_k = pl.pallas_call(
            _pre_ffn_body,
            grid=(CBLK,),
            in_specs=[
                pl.BlockSpec((ROW_BLK, D), functools.partial(lambda k_, i: (k_ * CBLK + i, 0), k)),
                pl.BlockSpec((D, D), lambda i: (0, 0)),
                pl.BlockSpec((1, D), lambda i: (0, 0)),
                pl.BlockSpec((D, D), lambda i: (0, 0)),
                pl.BlockSpec((1, D), lambda i: (0, 0)),
            ],
            out_specs=[
                pl.BlockSpec((ROW_BLK, D), lambda i: (i, 0)),
                pl.BlockSpec((1, GPB, D), lambda i: (i, 0, 0)),
            ],
            out_shape=[
                jax.ShapeDtypeStruct((CROWS, D), jnp.float32),
                jax.ShapeDtypeStruct((CBLK + 2, GPB, D), jnp.float32),
            ],
        )(x, w1_bf, b1p, w2_bf, b2p)
        gs_flat = gs_k.reshape((CBLK + 2) * GPB, D)
        p0, p1 = sc_call(idx3[k], ftx_k, gs_flat, tgt3[k], bnd3[k])
        partials.append(p0)
        partials.append(p1)

    pstack = jnp.stack(partials, axis=0)

    out = pl.pallas_call(
        _post_ffn_body,
        in_specs=[
            pl.BlockSpec((2 * CH, NUM_SEGMENTS, D), lambda: (0, 0, 0)),
            pl.BlockSpec((D, D), lambda: (0, 0)),
            pl.BlockSpec((1, D), lambda: (0, 0)),
            pl.BlockSpec((D, G), lambda: (0, 0)),
            pl.BlockSpec((1, G), lambda: (0, 0)),
        ],
        out_specs=pl.BlockSpec((NUM_SEGMENTS, G), lambda: (0, 0)),
        out_shape=jax.ShapeDtypeStruct((NUM_SEGMENTS, G), jnp.float32),
    )(pstack, W1_post, b1q, W2_post, b2q)
    return out


# speedup vs baseline: 1.4807x; 1.1677x over previous
import functools

import jax
import jax.numpy as jnp
from jax import lax
from jax.experimental import pallas as pl
from jax.experimental.pallas import tpu as pltpu
from jax.experimental.pallas import tpu_sc as plsc

N = 320000
D = 128
G = 64
NUM_SEGMENTS = 256

GRP = 128
NGRP = N // GRP
CH = 2
CGRP = NGRP // CH
CROWS = N // CH
ROW_BLK = 6400
GPB = ROW_BLK // GRP
CBLK = CROWS // ROW_BLK
NW = 32
PADG = ((CGRP // NW + 1 + 7 + 15) // 16) * 16
TRASH = NUM_SEGMENTS


def _pre_ffn_body(x_ref, w1_ref, b1_ref, w2_ref, b2_ref, out_ref, gs_ref):
    xb = x_ref[...].astype(jnp.bfloat16)
    h = jnp.maximum(
        lax.dot_general(xb, w1_ref[...], (((1,), (0,)), ((), ())),
                        preferred_element_type=jnp.float32) + b1_ref[...],
        0.0)
    ftx = lax.dot_general(h.astype(jnp.bfloat16), w2_ref[...],
                          (((1,), (0,)), ((), ())),
                          preferred_element_type=jnp.float32) + b2_ref[...]
    out_ref[...] = ftx
    gs_ref[0] = jnp.sum(ftx.reshape(GPB, GRP, D), axis=1)


def _post_ffn_body(p_ref, w1_ref, b1_ref, w2_ref, b2_ref, out_ref):
    g = jnp.sum(p_ref[...], axis=0)
    h = jnp.maximum(
        lax.dot_general(g, w1_ref[...], (((1,), (0,)), ((), ())),
                        preferred_element_type=jnp.float32) + b1_ref[...],
        0.0)
    out_ref[...] = lax.dot_general(h, w2_ref[...], (((1,), (0,)), ((), ())),
                                   preferred_element_type=jnp.float32) + b2_ref[...]


def _sc_chunk_body(idx3, ftx, gsums, tgt3, bnd3, out0, out1,
                   idx_v, gsum_v, tgt_v, bnd_v, rows_b, zeros_v, acc_sh,
                   semi, semg, semt):
    nc = lax.axis_size("c")
    ns = lax.axis_size("s")
    c = lax.axis_index("c")
    s = lax.axis_index("s")
    w = c * ns + s

    g0a = pl.multiple_of(((CGRP * w) // NW // 8) * 8, 8)

    pltpu.async_copy(idx3.at[w], idx_v, semi)
    pltpu.async_copy(gsums.at[pl.ds(g0a, PADG)], gsum_v, semg)
    pltpu.async_copy(tgt3.at[w], tgt_v, semt)
    pltpu.async_copy(bnd3.at[w], bnd_v, semt)

    for i in range(16):
        for j in range(D // 16):
            zeros_v[i, pl.ds(j * 16, 16)] = jnp.zeros((16,), jnp.float32)
    pltpu.sync_copy(zeros_v, acc_sh.at[pl.ds(s * 16, 16)])

    @pl.when(s == 0)
    def _():
        pltpu.sync_copy(zeros_v.at[pl.ds(0, 1)], acc_sh.at[pl.ds(256, 1)])

    pltpu.make_async_copy(idx3.at[0], idx_v, semi).wait()
    pltpu.make_async_copy(tgt3.at[0], tgt_v, semt).wait()
    pltpu.make_async_copy(bnd3.at[0], bnd_v, semt).wait()
    pltpu.make_async_copy(gsums.at[pl.ds(0, PADG)], gsum_v, semg).wait()
    plsc.subcore_barrier()

    pltpu.sync_copy(gsum_v, acc_sh.at[tgt_v], add=True)

    def body(j, carry):
        flag = bnd_v[pl.ds(j, 16)][0]

        @pl.when(flag > 0)
        def _():
            pltpu.sync_copy(ftx.at[pl.ds((g0a + j) * GRP, GRP)], rows_b)
            pltpu.sync_copy(rows_b, acc_sh.at[idx_v.at[j]], add=True)
        return carry

    lax.fori_loop(0, PADG, body, 0)
    plsc.subcore_barrier()

    @pl.when((s == 0) & (c == 0))
    def _():
        pltpu.sync_copy(acc_sh.at[pl.ds(0, NUM_SEGMENTS)], out0)

    @pl.when((s == 0) & (c == 1))
    def _():
        pltpu.sync_copy(acc_sh.at[pl.ds(0, NUM_SEGMENTS)], out1)


def kernel(x, batch, W1_pre, b1_pre, W2_pre, b2_pre, W1_post, b1_post, W2_post, b2_post):
    batch2d = batch.astype(jnp.int32).reshape(NGRP, GRP)
    b1p = b1_pre.reshape(1, D)
    b2p = b2_pre.reshape(1, D)
    b1q = b1_post.reshape(1, D)
    b2q = b2_post.reshape(1, G)
    w1_bf = W1_pre.astype(jnp.bfloat16)
    w2_bf = W2_pre.astype(jnp.bfloat16)

    g0s = (CGRP * jnp.arange(NW, dtype=jnp.int32)) // NW
    g1s = (CGRP * (jnp.arange(NW, dtype=jnp.int32) + 1)) // NW
    g0as = (g0s // 8) * 8
    slabrows = jnp.minimum(
        g0as[:, None] + jnp.arange(PADG, dtype=jnp.int32)[None, :], CGRP - 1)
    crows = (slabrows[None, :, :]
             + CGRP * jnp.arange(CH, dtype=jnp.int32)[:, None, None])
    idx3 = batch2d[crows.reshape(-1)].reshape(CH, NW, PADG, GRP)
    firsts = batch2d[:, 0]
    lasts = batch2d[:, GRP - 1]
    uni = firsts == lasts
    tgt_global = jnp.where(uni, firsts, TRASH).astype(jnp.int32)
    slot = jnp.arange(PADG, dtype=jnp.int32)[None, :]
    validP = ((slot >= (g0s - g0as)[:, None])
              & (slot < (g1s - g0as)[:, None]))
    tgt3 = jnp.where(validP[None], tgt_global[crows.reshape(-1)].reshape(
        CH, NW, PADG), TRASH).astype(jnp.int32)
    bnd3 = jnp.where(validP[None] & ~uni[crows.reshape(-1)].reshape(
        CH, NW, PADG), 1, 0).astype(jnp.int32)
    bnd3 = jnp.concatenate(
        [bnd3, jnp.zeros((CH, NW, 16), jnp.int32)], axis=-1)

    mesh = plsc.VectorSubcoreMesh(core_axis_name="c", subcore_axis_name="s",
                                  num_cores=2, num_subcores=16)
    sc_call = pl.kernel(
        _sc_chunk_body,
        out_type=[jax.ShapeDtypeStruct((NUM_SEGMENTS, D), jnp.float32),
                  jax.ShapeDtypeStruct((NUM_SEGMENTS, D), jnp.float32)],
        mesh=mesh,
        scratch_types=[
            pltpu.VMEM((PADG, GRP), jnp.int32),
            pltpu.VMEM((PADG, D), jnp.float32),
            pltpu.VMEM((PADG,), jnp.int32),
            pltpu.VMEM((PADG + 16,), jnp.int32),
            pltpu.VMEM((GRP, D), jnp.float32),
            pltpu.VMEM((16, D), jnp.float32),
            pltpu.VMEM_SHARED((NUM_SEGMENTS + 1, D), jnp.float32),
            pltpu.SemaphoreType.DMA,
            pltpu.SemaphoreType.DMA,
            pltpu.SemaphoreType.DMA,
        ],
    )

    partials = []
    for k in range(CH):
        ftx_k, gs_k = pl.pallas_call(
            _pre_ffn_body,
            grid=(CBLK,),
            in_specs=[
                pl.BlockSpec((ROW_BLK, D), functools.partial(lambda k_, i: (k_ * CBLK + i, 0), k)),
                pl.BlockSpec((D, D), lambda i: (0, 0)),
                pl.BlockSpec((1, D), lambda i: (0, 0)),
                pl.BlockSpec((D, D), lambda i: (0, 0)),
                pl.BlockSpec((1, D), lambda i: (0, 0)),
            ],
            out_specs=[
                pl.BlockSpec((ROW_BLK, D), lambda i: (i, 0)),
                pl.BlockSpec((1, GPB, D), lambda i: (i, 0, 0)),
            ],
            out_shape=[
                jax.ShapeDtypeStruct((CROWS, D), jnp.float32),
                jax.ShapeDtypeStruct((CBLK + 2, GPB, D), jnp.float32),
            ],
        )(x, w1_bf, b1p, w2_bf, b2p)
        gs_flat = gs_k.reshape((CBLK + 2) * GPB, D)
        p0, p1 = sc_call(idx3[k], ftx_k, gs_flat, tgt3[k], bnd3[k])
        partials.append(p0)
        partials.append(p1)

    pstack = jnp.stack(partials, axis=0)

    out = pl.pallas_call(
        _post_ffn_body,
        in_specs=[
            pl.BlockSpec((2 * CH, NUM_SEGMENTS, D), lambda: (0, 0, 0)),
            pl.BlockSpec((D, D), lambda: (0, 0)),
            pl.BlockSpec((1, D), lambda: (0, 0)),
            pl.BlockSpec((D, G), lambda: (0, 0)),
            pl.BlockSpec((1, G), lambda: (0, 0)),
        ],
        out_specs=pl.BlockSpec((NUM_SEGMENTS, G), lambda: (0, 0)),
        out_shape=jax.ShapeDtypeStruct((NUM_SEGMENTS, G), jnp.float32),
    )(pstack, W1_post, b1q, W2_post, b2q)
    return out


# speedup vs baseline: 1.5525x; 1.0485x over previous
import functools

import jax
import jax.numpy as jnp
from jax import lax
from jax.experimental import pallas as pl
from jax.experimental.pallas import tpu as pltpu
from jax.experimental.pallas import tpu_sc as plsc

N = 320000
D = 128
G = 64
NUM_SEGMENTS = 256

GRP = 128
NGRP = N // GRP
CH = 2
CGRP = NGRP // CH
CROWS = N // CH
ROW_BLK = 16000
GPB = ROW_BLK // GRP
CBLK = CROWS // ROW_BLK
NW = 32
PADG = ((CGRP // NW + 1 + 7 + 15) // 16) * 16
TRASH = NUM_SEGMENTS


def _pre_ffn_body(x_ref, w1_ref, b1_ref, w2_ref, b2_ref, out_ref, gs_ref):
    xb = x_ref[...].astype(jnp.bfloat16)
    h = jnp.maximum(
        lax.dot_general(xb, w1_ref[...], (((1,), (0,)), ((), ())),
                        preferred_element_type=jnp.float32) + b1_ref[...],
        0.0)
    ftx = lax.dot_general(h.astype(jnp.bfloat16), w2_ref[...],
                          (((1,), (0,)), ((), ())),
                          preferred_element_type=jnp.float32) + b2_ref[...]
    out_ref[...] = ftx
    gs_ref[0] = jnp.sum(ftx.reshape(GPB, GRP, D), axis=1)


def _post_ffn_body(p_ref, w1_ref, b1_ref, w2_ref, b2_ref, out_ref):
    g = jnp.sum(p_ref[...], axis=0)
    h = jnp.maximum(
        lax.dot_general(g, w1_ref[...], (((1,), (0,)), ((), ())),
                        preferred_element_type=jnp.float32) + b1_ref[...],
        0.0)
    out_ref[...] = lax.dot_general(h, w2_ref[...], (((1,), (0,)), ((), ())),
                                   preferred_element_type=jnp.float32) + b2_ref[...]


def _sc_chunk_body(idx3, ftx, gsums, tgt3, bnd3, out0, out1,
                   idx_v, gsum_v, tgt_v, bnd_v, rows_b, zeros_v, acc_sh,
                   semi, semg, semt):
    nc = lax.axis_size("c")
    ns = lax.axis_size("s")
    c = lax.axis_index("c")
    s = lax.axis_index("s")
    w = c * ns + s

    g0a = pl.multiple_of(((CGRP * w) // NW // 8) * 8, 8)

    pltpu.async_copy(idx3.at[w], idx_v, semi)
    pltpu.async_copy(gsums.at[pl.ds(g0a, PADG)], gsum_v, semg)
    pltpu.async_copy(tgt3.at[w], tgt_v, semt)
    pltpu.async_copy(bnd3.at[w], bnd_v, semt)

    for i in range(16):
        for j in range(D // 16):
            zeros_v[i, pl.ds(j * 16, 16)] = jnp.zeros((16,), jnp.float32)
    pltpu.sync_copy(zeros_v, acc_sh.at[pl.ds(s * 16, 16)])

    @pl.when(s == 0)
    def _():
        pltpu.sync_copy(zeros_v.at[pl.ds(0, 1)], acc_sh.at[pl.ds(256, 1)])

    pltpu.make_async_copy(idx3.at[0], idx_v, semi).wait()
    pltpu.make_async_copy(tgt3.at[0], tgt_v, semt).wait()
    pltpu.make_async_copy(bnd3.at[0], bnd_v, semt).wait()
    pltpu.make_async_copy(gsums.at[pl.ds(0, PADG)], gsum_v, semg).wait()
    plsc.subcore_barrier()

    pltpu.sync_copy(gsum_v, acc_sh.at[tgt_v], add=True)

    def body(j, carry):
        flag = bnd_v[pl.ds(j, 16)][0]

        @pl.when(flag > 0)
        def _():
            pltpu.sync_copy(ftx.at[pl.ds((g0a + j) * GRP, GRP)], rows_b)
            pltpu.sync_copy(rows_b, acc_sh.at[idx_v.at[j]], add=True)
        return carry

    lax.fori_loop(0, PADG, body, 0)
    plsc.subcore_barrier()

    @pl.when((s == 0) & (c == 0))
    def _():
        pltpu.sync_copy(acc_sh.at[pl.ds(0, NUM_SEGMENTS)], out0)

    @pl.when((s == 0) & (c == 1))
    def _():
        pltpu.sync_copy(acc_sh.at[pl.ds(0, NUM_SEGMENTS)], out1)


def kernel(x, batch, W1_pre, b1_pre, W2_pre, b2_pre, W1_post, b1_post, W2_post, b2_post):
    batch2d = batch.astype(jnp.int32).reshape(NGRP, GRP)
    b1p = b1_pre.reshape(1, D)
    b2p = b2_pre.reshape(1, D)
    b1q = b1_post.reshape(1, D)
    b2q = b2_post.reshape(1, G)
    w1_bf = W1_pre.astype(jnp.bfloat16)
    w2_bf = W2_pre.astype(jnp.bfloat16)

    g0s = (CGRP * jnp.arange(NW, dtype=jnp.int32)) // NW
    g1s = (CGRP * (jnp.arange(NW, dtype=jnp.int32) + 1)) // NW
    g0as = (g0s // 8) * 8
    slabrows = jnp.minimum(
        g0as[:, None] + jnp.arange(PADG, dtype=jnp.int32)[None, :], CGRP - 1)
    crows = (slabrows[None, :, :]
             + CGRP * jnp.arange(CH, dtype=jnp.int32)[:, None, None])
    idx3 = batch2d[crows.reshape(-1)].reshape(CH, NW, PADG, GRP)
    firsts = batch2d[:, 0]
    lasts = batch2d[:, GRP - 1]
    uni = firsts == lasts
    tgt_global = jnp.where(uni, firsts, TRASH).astype(jnp.int32)
    slot = jnp.arange(PADG, dtype=jnp.int32)[None, :]
    validP = ((slot >= (g0s - g0as)[:, None])
              & (slot < (g1s - g0as)[:, None]))
    tgt3 = jnp.where(validP[None], tgt_global[crows.reshape(-1)].reshape(
        CH, NW, PADG), TRASH).astype(jnp.int32)
    bnd3 = jnp.where(validP[None] & ~uni[crows.reshape(-1)].reshape(
        CH, NW, PADG), 1, 0).astype(jnp.int32)
    bnd3 = jnp.concatenate(
        [bnd3, jnp.zeros((CH, NW, 16), jnp.int32)], axis=-1)

    mesh = plsc.VectorSubcoreMesh(core_axis_name="c", subcore_axis_name="s",
                                  num_cores=2, num_subcores=16)
    sc_call = pl.kernel(
        _sc_chunk_body,
        out_type=[jax.ShapeDtypeStruct((NUM_SEGMENTS, D), jnp.float32),
                  jax.ShapeDtypeStruct((NUM_SEGMENTS, D), jnp.float32)],
        mesh=mesh,
        scratch_types=[
            pltpu.VMEM((PADG, GRP), jnp.int32),
            pltpu.VMEM((PADG, D), jnp.float32),
            pltpu.VMEM((PADG,), jnp.int32),
            pltpu.VMEM((PADG + 16,), jnp.int32),
            pltpu.VMEM((GRP, D), jnp.float32),
            pltpu.VMEM((16, D), jnp.float32),
            pltpu.VMEM_SHARED((NUM_SEGMENTS + 1, D), jnp.float32),
            pltpu.SemaphoreType.DMA,
            pltpu.SemaphoreType.DMA,
            pltpu.SemaphoreType.DMA,
        ],
    )

    partials = []
    for k in range(CH):
        ftx_k, gs_k = pl.pallas_call(
            _pre_ffn_body,
            grid=(CBLK,),
            in_specs=[
                pl.BlockSpec((ROW_BLK, D), functools.partial(lambda k_, i: (k_ * CBLK + i, 0), k)),
                pl.BlockSpec((D, D), lambda i: (0, 0)),
                pl.BlockSpec((1, D), lambda i: (0, 0)),
                pl.BlockSpec((D, D), lambda i: (0, 0)),
                pl.BlockSpec((1, D), lambda i: (0, 0)),
            ],
            out_specs=[
                pl.BlockSpec((ROW_BLK, D), lambda i: (i, 0)),
                pl.BlockSpec((1, GPB, D), lambda i: (i, 0, 0)),
            ],
            out_shape=[
                jax.ShapeDtypeStruct((CROWS, D), jnp.float32),
                jax.ShapeDtypeStruct((CBLK + 2, GPB, D), jnp.float32),
            ],
        )(x, w1_bf, b1p, w2_bf, b2p)
        gs_flat = gs_k.reshape((CBLK + 2) * GPB, D)
        p0, p1 = sc_call(idx3[k], ftx_k, gs_flat, tgt3[k], bnd3[k])
        partials.append(p0)
        partials.append(p1)

    pstack = jnp.stack(partials, axis=0)

    out = pl.pallas_call(
        _post_ffn_body,
        in_specs=[
            pl.BlockSpec((2 * CH, NUM_SEGMENTS, D), lambda: (0, 0, 0)),
            pl.BlockSpec((D, D), lambda: (0, 0)),
            pl.BlockSpec((1, D), lambda: (0, 0)),
            pl.BlockSpec((D, G), lambda: (0, 0)),
            pl.BlockSpec((1, G), lambda: (0, 0)),
        ],
        out_specs=pl.BlockSpec((NUM_SEGMENTS, G), lambda: (0, 0)),
        out_shape=jax.ShapeDtypeStruct((NUM_SEGMENTS, G), jnp.float32),
    )(pstack, W1_post, b1q, W2_post, b2q)
    return out
